# Initial kernel scaffold; baseline (speedup 1.0000x reference)
import jax
import jax.numpy as jnp
from jax.experimental import pallas as pl


def kernel(x, edge_index, W1, b1, W2, b2):
    raise NotImplementedError("write your pallas kernel here")


# speedup vs baseline: 1.0295x; 1.0295x over previous
import functools

import jax
import jax.numpy as jnp
from jax.experimental import pallas as pl

FEATURE_SIZE = 128
FP_LENGTH = 512
RADIUS = 3
N_NODES = 10000
N_EDGES = 320000

_BR = 1000


def _dense_body(feats_ref, agg_ref, w1_ref, b1_ref, w2_ref, b2_ref,
                h_ref, fp_ref):
    i = pl.program_id(0)
    ns = feats_ref[...] + agg_ref[...]
    h = jax.lax.dot_general(ns, w1_ref[...], (((1,), (1,)), ((), ())),
                            preferred_element_type=jnp.float32)
    h = jnp.maximum(h + b1_ref[...], 0.0)
    h_ref[...] = h
    logits = jax.lax.dot_general(h, w2_ref[...], (((1,), (1,)), ((), ())),
                                 preferred_element_type=jnp.float32)
    logits = logits + b2_ref[...]
    m = jnp.max(logits, axis=1, keepdims=True)
    e = jnp.exp(logits - m)
    p = e / jnp.sum(e, axis=1, keepdims=True)
    part = jnp.sum(p, axis=0, keepdims=True)

    @pl.when(i == 0)
    def _():
        fp_ref[...] = part

    @pl.when(i != 0)
    def _():
        fp_ref[...] = fp_ref[...] + part


def _make_dense(interpret=False):
    grid = (N_NODES // _BR,)
    return pl.pallas_call(
        _dense_body,
        grid=grid,
        in_specs=[
            pl.BlockSpec((_BR, FEATURE_SIZE), lambda i: (i, 0)),
            pl.BlockSpec((_BR, FEATURE_SIZE), lambda i: (i, 0)),
            pl.BlockSpec((FEATURE_SIZE, FEATURE_SIZE), lambda i: (0, 0)),
            pl.BlockSpec((1, FEATURE_SIZE), lambda i: (0, 0)),
            pl.BlockSpec((FP_LENGTH, FEATURE_SIZE), lambda i: (0, 0)),
            pl.BlockSpec((1, FP_LENGTH), lambda i: (0, 0)),
        ],
        out_specs=[
            pl.BlockSpec((_BR, FEATURE_SIZE), lambda i: (i, 0)),
            pl.BlockSpec((1, FP_LENGTH), lambda i: (0, 0)),
        ],
        out_shape=[
            jax.ShapeDtypeStruct((N_NODES, FEATURE_SIZE), jnp.float32),
            jax.ShapeDtypeStruct((1, FP_LENGTH), jnp.float32),
        ],
        interpret=interpret,
    )


def kernel(x, edge_index, W1, b1, W2, b2, interpret=False):
    dense = _make_dense(interpret)
    src = edge_index[0].astype(jnp.int32)
    dst = edge_index[1].astype(jnp.int32)
    b1r = b1.reshape(1, FEATURE_SIZE)
    b2r = b2.reshape(1, FP_LENGTH)
    feats = x
    fp = jnp.zeros((1, FP_LENGTH), dtype=jnp.float32)
    for _ in range(RADIUS):
        gathered = jnp.take(feats, src, axis=0)
        agg = jax.ops.segment_sum(gathered, dst, num_segments=N_NODES)
        h, fp_part = dense(feats, agg, W1, b1r, W2, b2r)
        fp = fp + fp_part
        feats = h
    return fp


# speedup vs baseline: 8.2360x; 7.9996x over previous
import functools

import jax
import jax.numpy as jnp
from jax.experimental import pallas as pl
from jax.experimental.pallas import tpu as pltpu
from jax.experimental.pallas import tpu_sc as plsc

FEATURE_SIZE = 128
FP_LENGTH = 512
RADIUS = 3
N_NODES = 10000
N_EDGES = 320000

_BR = 1000

_NC, _NS, _L = 2, 16, 16
_NW = _NC * _NS
_EPW = N_EDGES // _NW
_CHUNK = 125
_NCH = _EPW // _CHUNK
_NP = 10240
_RPT = _NP // _NS
_ZC = 80
_ZCH = _RPT // _ZC


def _sc_agg_body(x_hbm, src_hbm, dst_hbm, out_hbm,
                 sidx_v, didx_v, rows_v, agg_sh, sem):
    c = jax.lax.axis_index("c")
    s = jax.lax.axis_index("s")

    zero = jnp.zeros((_L,), jnp.float32)
    qpr = FEATURE_SIZE // _L

    def _zrow(i, carry):
        rows_v[i // qpr, pl.ds((i % qpr) * _L, _L)] = zero
        return carry

    jax.lax.fori_loop(0, _ZC * qpr, _zrow, 0)

    def _zcopy(k, carry):
        pltpu.sync_copy(rows_v.at[pl.ds(0, _ZC)],
                        agg_sh.at[pl.ds(s * _RPT + k * _ZC, _ZC)])
        return carry

    jax.lax.fori_loop(0, _ZCH, _zcopy, 0)
    plsc.subcore_barrier()

    pltpu.sync_copy(src_hbm.at[c, s], sidx_v)
    pltpu.sync_copy(dst_hbm.at[c, s], didx_v)

    def _step(j, carry):
        pltpu.async_copy(x_hbm.at[sidx_v.at[j]], rows_v, sem).wait()
        pltpu.sync_copy(rows_v, agg_sh.at[didx_v.at[j]], add=True)
        return carry

    jax.lax.fori_loop(0, _NCH, _step, 0)
    plsc.subcore_barrier()

    pltpu.sync_copy(agg_sh.at[pl.ds(s * _RPT, _RPT)],
                    out_hbm.at[c, pl.ds(s * _RPT, _RPT)])


_sc_agg = pl.kernel(
    _sc_agg_body,
    out_type=jax.ShapeDtypeStruct((_NC, _NP, FEATURE_SIZE), jnp.float32),
    mesh=plsc.VectorSubcoreMesh(core_axis_name="c", subcore_axis_name="s"),
    scratch_types=[
        pltpu.VMEM((_NCH, _CHUNK), jnp.int32),
        pltpu.VMEM((_NCH, _CHUNK), jnp.int32),
        pltpu.VMEM((_CHUNK, FEATURE_SIZE), jnp.float32),
        pltpu.VMEM_SHARED((_NP, FEATURE_SIZE), jnp.float32),
        pltpu.SemaphoreType.DMA,
    ],
)


def _dense_body(feats_ref, agg_ref, w1_ref, b1_ref, w2_ref, b2_ref,
                h_ref, fp_ref):
    i = pl.program_id(0)
    ns = feats_ref[...] + agg_ref[0] + agg_ref[1]
    h = jax.lax.dot_general(ns, w1_ref[...], (((1,), (1,)), ((), ())),
                            preferred_element_type=jnp.float32)
    h = jnp.maximum(h + b1_ref[...], 0.0)
    h_ref[...] = h
    logits = jax.lax.dot_general(h, w2_ref[...], (((1,), (1,)), ((), ())),
                                 preferred_element_type=jnp.float32)
    logits = logits + b2_ref[...]
    m = jnp.max(logits, axis=1, keepdims=True)
    e = jnp.exp(logits - m)
    p = e / jnp.sum(e, axis=1, keepdims=True)
    part = jnp.sum(p, axis=0, keepdims=True)

    @pl.when(i == 0)
    def _():
        fp_ref[...] = part

    @pl.when(i != 0)
    def _():
        fp_ref[...] = fp_ref[...] + part


def _make_dense(interpret=False):
    grid = (N_NODES // _BR,)
    return pl.pallas_call(
        _dense_body,
        grid=grid,
        in_specs=[
            pl.BlockSpec((_BR, FEATURE_SIZE), lambda i: (i, 0)),
            pl.BlockSpec((_NC, _BR, FEATURE_SIZE), lambda i: (0, i, 0)),
            pl.BlockSpec((FEATURE_SIZE, FEATURE_SIZE), lambda i: (0, 0)),
            pl.BlockSpec((1, FEATURE_SIZE), lambda i: (0, 0)),
            pl.BlockSpec((FP_LENGTH, FEATURE_SIZE), lambda i: (0, 0)),
            pl.BlockSpec((1, FP_LENGTH), lambda i: (0, 0)),
        ],
        out_specs=[
            pl.BlockSpec((_BR, FEATURE_SIZE), lambda i: (i, 0)),
            pl.BlockSpec((1, FP_LENGTH), lambda i: (0, 0)),
        ],
        out_shape=[
            jax.ShapeDtypeStruct((N_NODES, FEATURE_SIZE), jnp.float32),
            jax.ShapeDtypeStruct((1, FP_LENGTH), jnp.float32),
        ],
        interpret=interpret,
    )


def kernel(x, edge_index, W1, b1, W2, b2, interpret=False):
    dense = _make_dense(interpret)
    src = edge_index[0].astype(jnp.int32).reshape(_NC, _NS, _NCH, _CHUNK)
    dst = edge_index[1].astype(jnp.int32).reshape(_NC, _NS, _NCH, _CHUNK)
    b1r = b1.reshape(1, FEATURE_SIZE)
    b2r = b2.reshape(1, FP_LENGTH)
    feats = x
    fp = jnp.zeros((1, FP_LENGTH), dtype=jnp.float32)
    for _ in range(RADIUS):
        agg2 = _sc_agg(feats, src, dst)
        h, fp_part = dense(feats, agg2, W1, b1r, W2, b2r)
        fp = fp + fp_part
        feats = h
    return fp
